# baseline (device time: 101612 ns/iter reference)
import jax
import jax.numpy as jnp
from jax import lax
from jax.experimental import pallas as pl
from jax.experimental.pallas import tpu as pltpu

N_DEV = 16
B, SQ, D = 4, 256, 1024
SKV = 1024
H_LOC = 8
DH = 128
SCALE = 0.08838834764831843
CHUNK = SQ // N_DEV


def _fused_body(x_ref, wq_ref, wo_ref, k_ref, v_ref, o_ref,
                rs_recv, ag_recv, send_stage, ag_send, acc_ref, attn_ref,
                rs_ssems, rs_rsems, ag_ssems, ag_rsems):
    b = pl.program_id(0)
    h = pl.program_id(1)
    me = lax.axis_index("i")

    @pl.when((b == 0) & (h == 0))
    def _entry():
        barrier = pltpu.get_barrier_semaphore()
        for d in range(N_DEV):
            @pl.when(me != d)
            def _(d=d):
                pl.semaphore_signal(barrier, inc=1, device_id=(d,),
                                    device_id_type=pl.DeviceIdType.MESH)
        pl.semaphore_wait(barrier, N_DEV - 1)

    q = jnp.dot(x_ref[0], wq_ref[...], preferred_element_type=jnp.float32)
    qb = (q * SCALE).astype(jnp.bfloat16)
    s = jnp.dot(qb, k_ref[0].T, preferred_element_type=jnp.float32)
    p = jnp.exp(s)
    pb = p.astype(jnp.bfloat16)
    ones = jnp.ones((SKV, DH), jnp.bfloat16)
    lcol = jnp.dot(pb, ones, preferred_element_type=jnp.float32)[:, 0:1]
    o = jnp.dot(pb, v_ref[0], preferred_element_type=jnp.float32) / lcol
    attn_ref[:, pl.ds(h * DH, DH)] = o.astype(jnp.bfloat16)

    @pl.when(h == H_LOC - 1)
    def _():
        o_ref[b] = jnp.dot(attn_ref[...], wo_ref[...],
                           preferred_element_type=jnp.float32)

    def rs_desc(g, d):
        return pltpu.make_async_remote_copy(
            src_ref=send_stage.at[g, pl.ds(d * CHUNK, CHUNK), :],
            dst_ref=rs_recv.at[g, me],
            send_sem=rs_ssems.at[g, d],
            recv_sem=rs_rsems.at[g, me],
            device_id=(d,),
            device_id_type=pl.DeviceIdType.MESH,
        )

    def rs_mirror_desc(g, p):
        return pltpu.make_async_remote_copy(
            src_ref=send_stage.at[g, pl.ds(p * CHUNK, CHUNK), :],
            dst_ref=rs_recv.at[g, p],
            send_sem=rs_ssems.at[g, p],
            recv_sem=rs_rsems.at[g, p],
            device_id=(p,),
            device_id_type=pl.DeviceIdType.MESH,
        )

    def ag_desc(g, d):
        return pltpu.make_async_remote_copy(
            src_ref=ag_send.at[g],
            dst_ref=ag_recv.at[g, me],
            send_sem=ag_ssems.at[g, d],
            recv_sem=ag_rsems.at[g, me],
            device_id=(d,),
            device_id_type=pl.DeviceIdType.MESH,
        )

    def ag_mirror_desc(g, d):
        return pltpu.make_async_remote_copy(
            src_ref=ag_send.at[g],
            dst_ref=ag_recv.at[g, d],
            send_sem=ag_ssems.at[g, d],
            recv_sem=ag_rsems.at[g, d],
            device_id=(d,),
            device_id_type=pl.DeviceIdType.MESH,
        )

    def rs_issue(g):
        send_stage[g] = o_ref[g].astype(jnp.bfloat16)
        for d in range(N_DEV):
            @pl.when(me != d)
            def _(d=d):
                rs_desc(g, d).start()

    def rs_reduce_ag_issue(g):
        acc_ref[...] = o_ref[g, pl.ds(me * CHUNK, CHUNK), :]
        for p in range(N_DEV):
            @pl.when(me != p)
            def _(p=p):
                desc = rs_mirror_desc(g, p)
                desc.wait_recv()
                desc.wait_send()
                acc_ref[...] = acc_ref[...] + rs_recv[g, p].astype(jnp.float32)
        total = acc_ref[...]
        o_ref[g, pl.ds(me * CHUNK, CHUNK), :] = total
        ag_send[g] = total.astype(jnp.bfloat16)
        for d in range(N_DEV):
            @pl.when(me != d)
            def _(d=d):
                ag_desc(g, d).start()

    def ag_store(g):
        for d in range(N_DEV):
            @pl.when(me != d)
            def _(d=d):
                desc = ag_mirror_desc(g, d)
                desc.wait_recv()
                o_ref[g, pl.ds(d * CHUNK, CHUNK), :] = (
                    ag_recv[g, d].astype(jnp.float32))
                desc.wait_send()

    import os
    if os.environ.get("SKIP_AR"):
        return

    @pl.when((b > 0) & (h == 3))
    def _():
        rs_reduce_ag_issue(b - 1)

    @pl.when((b > 1) & (h == 5))
    def _():
        ag_store(b - 2)

    @pl.when(h == 7)
    def _():
        rs_issue(b)

    @pl.when((b == 3) & (h == 7))
    def _drain():
        ag_store(2)
        rs_reduce_ag_issue(3)
        ag_store(3)


def kernel(x, Wq, Wo, K_ext, V_ext):
    xb = x.astype(jnp.bfloat16)
    wqb = Wq.astype(jnp.bfloat16)
    wob = Wo.astype(jnp.bfloat16)
    kb = K_ext.reshape(B, SKV, H_LOC * DH).astype(jnp.bfloat16)
    vb = V_ext.reshape(B, SKV, H_LOC * DH).astype(jnp.bfloat16)

    return pl.pallas_call(
        _fused_body,
        grid=(B, H_LOC),
        in_specs=[
            pl.BlockSpec((1, SQ, D), lambda b, h: (b, 0, 0)),
            pl.BlockSpec((D, DH), lambda b, h: (0, h)),
            pl.BlockSpec((D, D), lambda b, h: (0, 0)),
            pl.BlockSpec((1, SKV, DH), lambda b, h: (b, 0, h)),
            pl.BlockSpec((1, SKV, DH), lambda b, h: (b, 0, h)),
        ],
        out_specs=pl.BlockSpec((B, SQ, D), lambda b, h: (0, 0, 0)),
        out_shape=jax.ShapeDtypeStruct((B, SQ, D), jnp.float32),
        scratch_shapes=[
            pltpu.VMEM((B, N_DEV, CHUNK, D), jnp.bfloat16),
            pltpu.VMEM((B, N_DEV, CHUNK, D), jnp.bfloat16),
            pltpu.VMEM((B, SQ, D), jnp.bfloat16),
            pltpu.VMEM((B, CHUNK, D), jnp.bfloat16),
            pltpu.VMEM((CHUNK, D), jnp.float32),
            pltpu.VMEM((SQ, D), jnp.bfloat16),
            pltpu.SemaphoreType.DMA((B, N_DEV)),
            pltpu.SemaphoreType.DMA((B, N_DEV)),
            pltpu.SemaphoreType.DMA((B, N_DEV)),
            pltpu.SemaphoreType.DMA((B, N_DEV)),
        ],
        compiler_params=pltpu.CompilerParams(collective_id=0),
    )(xb, wqb, wob, kb, vb)


# device time: 92339 ns/iter; 1.1004x vs baseline; 1.1004x over previous
import jax
import jax.numpy as jnp
from jax import lax
from jax.experimental import pallas as pl
from jax.experimental.pallas import tpu as pltpu

N_DEV = 16
B, SQ, D = 4, 256, 1024
SKV = 1024
H_LOC = 8
DH = 128
SCALE = 0.08838834764831843
CHUNK = SQ // N_DEV


def _fused_body(x_ref, wq_ref, wo_ref, k_ref, v_ref, o_ref,
                rs_recv, ag_recv, send_stage, ag_send, acc_ref, attn_ref,
                rs_ssems, rs_rsems, ag_ssems, ag_rsems):
    b = pl.program_id(0)
    me = lax.axis_index("i")

    @pl.when(b == 0)
    def _entry():
        barrier = pltpu.get_barrier_semaphore()
        for d in range(N_DEV):
            @pl.when(me != d)
            def _(d=d):
                pl.semaphore_signal(barrier, inc=1, device_id=(d,),
                                    device_id_type=pl.DeviceIdType.MESH)
        pl.semaphore_wait(barrier, N_DEV - 1)

    xb = x_ref[0]
    q_all = jnp.dot(xb, wq_ref[...], preferred_element_type=jnp.float32)
    qb_all = (q_all * (SCALE * 1.4426950408889634)).astype(jnp.bfloat16)
    ones = jnp.ones((SKV, DH), jnp.bfloat16)
    for h in range(H_LOC):
        sl = slice(h * DH, (h + 1) * DH)
        qh = qb_all[:, sl]
        kh = k_ref[0, :, sl]
        s = jnp.dot(qh, kh.T, preferred_element_type=jnp.float32)
        pb = jnp.exp2(s).astype(jnp.bfloat16)
        lcol = jnp.dot(pb, ones, preferred_element_type=jnp.float32)[:, 0:1]
        o = jnp.dot(pb, v_ref[0, :, sl],
                    preferred_element_type=jnp.float32) / lcol
        attn_ref[:, sl] = o.astype(jnp.bfloat16)

    o_ref[b] = jnp.dot(attn_ref[...], wo_ref[...],
                       preferred_element_type=jnp.float32)

    def rs_desc(g, d):
        return pltpu.make_async_remote_copy(
            src_ref=send_stage.at[g, pl.ds(d * CHUNK, CHUNK), :],
            dst_ref=rs_recv.at[g, me],
            send_sem=rs_ssems.at[g, d],
            recv_sem=rs_rsems.at[g, me],
            device_id=(d,),
            device_id_type=pl.DeviceIdType.MESH,
        )

    def rs_mirror_desc(g, p):
        return pltpu.make_async_remote_copy(
            src_ref=send_stage.at[g, pl.ds(p * CHUNK, CHUNK), :],
            dst_ref=rs_recv.at[g, p],
            send_sem=rs_ssems.at[g, p],
            recv_sem=rs_rsems.at[g, p],
            device_id=(p,),
            device_id_type=pl.DeviceIdType.MESH,
        )

    def ag_desc(g, d):
        return pltpu.make_async_remote_copy(
            src_ref=ag_send.at[g],
            dst_ref=ag_recv.at[g, me],
            send_sem=ag_ssems.at[g, d],
            recv_sem=ag_rsems.at[g, me],
            device_id=(d,),
            device_id_type=pl.DeviceIdType.MESH,
        )

    def ag_mirror_desc(g, d):
        return pltpu.make_async_remote_copy(
            src_ref=ag_send.at[g],
            dst_ref=ag_recv.at[g, d],
            send_sem=ag_ssems.at[g, d],
            recv_sem=ag_rsems.at[g, d],
            device_id=(d,),
            device_id_type=pl.DeviceIdType.MESH,
        )

    def rs_issue(g):
        send_stage[g] = o_ref[g].astype(jnp.bfloat16)
        for d in range(N_DEV):
            @pl.when(me != d)
            def _(d=d):
                rs_desc(g, d).start()

    def rs_reduce_ag_issue(g):
        acc_ref[...] = o_ref[g, pl.ds(me * CHUNK, CHUNK), :]
        for p in range(N_DEV):
            @pl.when(me != p)
            def _(p=p):
                desc = rs_mirror_desc(g, p)
                desc.wait_recv()
                desc.wait_send()
                acc_ref[...] = acc_ref[...] + rs_recv[g, p].astype(jnp.float32)
        total = acc_ref[...]
        o_ref[g, pl.ds(me * CHUNK, CHUNK), :] = total
        ag_send[g] = total.astype(jnp.bfloat16)
        for d in range(N_DEV):
            @pl.when(me != d)
            def _(d=d):
                ag_desc(g, d).start()

    def ag_store(g):
        for d in range(N_DEV):
            @pl.when(me != d)
            def _(d=d):
                desc = ag_mirror_desc(g, d)
                desc.wait_recv()
                o_ref[g, pl.ds(d * CHUNK, CHUNK), :] = (
                    ag_recv[g, d].astype(jnp.float32))
                desc.wait_send()

    rs_issue(b)

    @pl.when(b > 1)
    def _():
        ag_store(b - 2)

    @pl.when(b > 0)
    def _():
        rs_reduce_ag_issue(b - 1)

    @pl.when(b == 3)
    def _drain():
        rs_reduce_ag_issue(3)
        ag_store(2)
        ag_store(3)


def kernel(x, Wq, Wo, K_ext, V_ext):
    xb = x.astype(jnp.bfloat16)
    wqb = Wq.astype(jnp.bfloat16)
    wob = Wo.astype(jnp.bfloat16)
    kb = K_ext.reshape(B, SKV, H_LOC * DH).astype(jnp.bfloat16)
    vb = V_ext.reshape(B, SKV, H_LOC * DH).astype(jnp.bfloat16)

    return pl.pallas_call(
        _fused_body,
        grid=(B,),
        in_specs=[
            pl.BlockSpec((1, SQ, D), lambda b: (b, 0, 0)),
            pl.BlockSpec((D, D), lambda b: (0, 0)),
            pl.BlockSpec((D, D), lambda b: (0, 0)),
            pl.BlockSpec((1, SKV, H_LOC * DH), lambda b: (b, 0, 0)),
            pl.BlockSpec((1, SKV, H_LOC * DH), lambda b: (b, 0, 0)),
        ],
        out_specs=pl.BlockSpec((B, SQ, D), lambda b: (0, 0, 0)),
        out_shape=jax.ShapeDtypeStruct((B, SQ, D), jnp.float32),
        scratch_shapes=[
            pltpu.VMEM((B, N_DEV, CHUNK, D), jnp.bfloat16),
            pltpu.VMEM((B, N_DEV, CHUNK, D), jnp.bfloat16),
            pltpu.VMEM((B, SQ, D), jnp.bfloat16),
            pltpu.VMEM((B, CHUNK, D), jnp.bfloat16),
            pltpu.VMEM((CHUNK, D), jnp.float32),
            pltpu.VMEM((SQ, D), jnp.bfloat16),
            pltpu.SemaphoreType.DMA((B, N_DEV)),
            pltpu.SemaphoreType.DMA((B, N_DEV)),
            pltpu.SemaphoreType.DMA((B, N_DEV)),
            pltpu.SemaphoreType.DMA((B, N_DEV)),
        ],
        compiler_params=pltpu.CompilerParams(collective_id=0),
    )(xb, wqb, wob, kb, vb)
